# Initial kernel scaffold; baseline (speedup 1.0000x reference)
import jax
import jax.numpy as jnp
from jax.experimental import pallas as pl


def kernel(logits_s, logits_w, y_tilde, i, y_tilde_all, y_hat, T):
    raise NotImplementedError("write your pallas kernel here")


# speedup vs baseline: 18.7931x; 18.7931x over previous
import functools

import jax
import jax.numpy as jnp
from jax.experimental import pallas as pl

_C = 10
_TEMP_INV = 2.0
_THRESH = 0.95


def _body(lsT_ref, lwT_ref, yt_ref, ytall_ref, yhat_ref, t_ref, out_ref):
    j = pl.program_id(0)
    f32 = jnp.float32
    C = _C

    ytall = ytall_ref[...]
    yhat = yhat_ref[...]
    n = ytall.shape[1]
    c10 = jax.lax.broadcasted_iota(jnp.int32, (C, n), 0)
    c11 = jax.lax.broadcasted_iota(jnp.int32, (C + 1, n), 0)
    oh_yt = (ytall == c10).astype(f32)
    oh_yh = (yhat == c11).astype(f32)
    yy0 = jax.lax.dot_general(oh_yt, oh_yh, (((1,), (1,)), ((), ())),
                              preferred_element_type=f32)
    ones_row = jnp.ones((1, n), dtype=f32)
    y_dist = jax.lax.dot_general(ones_row, oh_yt, (((1,), (1,)), ((), ())),
                                 preferred_element_type=f32) / n
    yy = yy0[:, :C] + yy0[:, C:C + 1] * y_dist
    yy = yy / jnp.sum(yy, axis=0, keepdims=True)
    inv_yy = 1.0 / yy
    counts = jnp.sum(oh_yh, axis=1, keepdims=True)
    beta = counts / jnp.max(counts)
    beta = beta / (2.0 - beta)
    thr_tab = _THRESH * beta[:C, :]

    yt = yt_ref[0]
    lw = lwT_ref[...]
    ls = lsT_ref[...]
    blk = lw.shape[1]

    k10 = jax.lax.broadcasted_iota(jnp.int32, (C, blk), 0)
    oh = (yt == k10).astype(f32)
    t_gather = jax.lax.dot_general(t_ref[...], oh, (((1,), (0,)), ((), ())),
                                   preferred_element_type=f32)
    yy_gather = jax.lax.dot_general(inv_yy, oh, (((0,), (0,)), ((), ())),
                                    preferred_element_type=f32)
    w = t_gather * yy_gather

    x = lw * _TEMP_INV
    xm = jnp.max(x, axis=0, keepdims=True)
    e = jnp.exp(x - xm) * w
    s = jnp.sum(e, axis=0, keepdims=True)
    m = jnp.max(e, axis=0, keepdims=True)
    maxp = m / s
    cand = jnp.where(e == m, k10, C + 127)
    t = jnp.min(cand, axis=0, keepdims=True)
    oht = (t == k10).astype(f32)

    lm = jnp.max(ls, axis=0, keepdims=True)
    z = jnp.sum(jnp.exp(ls - lm), axis=0, keepdims=True)
    logz = lm + jnp.log(z)
    picked = jnp.sum(oht * ls, axis=0, keepdims=True)
    ce = logz - picked

    thr = jnp.sum(oht * thr_tab, axis=0, keepdims=True)
    contrib = jnp.where(maxp > thr, ce, 0.0)

    @pl.when(j == 0)
    def _init():
        out_ref[...] = jnp.zeros((1, 1), jnp.float32)

    out_ref[...] += jnp.sum(contrib, axis=1, keepdims=True)


@functools.partial(jax.jit, static_argnames=())
def kernel(logits_s, logits_w, y_tilde, i, y_tilde_all, y_hat, T):
    del i
    B, C = logits_s.shape
    N = y_tilde_all.shape[0]
    blk = 2048
    nb = B // blk

    lsT = jnp.transpose(logits_s).astype(jnp.float32)
    lwT = jnp.transpose(logits_w).astype(jnp.float32)
    yt3 = y_tilde.astype(jnp.int32).reshape(nb, 1, blk)
    ytall2 = y_tilde_all.astype(jnp.int32).reshape(1, N)
    yhat2 = y_hat.astype(jnp.int32).reshape(1, N)

    out = pl.pallas_call(
        _body,
        grid=(nb,),
        in_specs=[
            pl.BlockSpec((C, blk), lambda j: (0, j)),
            pl.BlockSpec((C, blk), lambda j: (0, j)),
            pl.BlockSpec((1, 1, blk), lambda j: (j, 0, 0)),
            pl.BlockSpec((1, N), lambda j: (0, 0)),
            pl.BlockSpec((1, N), lambda j: (0, 0)),
            pl.BlockSpec((C, C), lambda j: (0, 0)),
        ],
        out_specs=pl.BlockSpec((1, 1), lambda j: (0, 0)),
        out_shape=jax.ShapeDtypeStruct((1, 1), jnp.float32),
    )(lsT, lwT, yt3, ytall2, yhat2, T.astype(jnp.float32))
    return out[0, 0] / B
